# baseline (speedup 1.0000x reference)
import functools

import jax
import jax.numpy as jnp
from jax import lax
from jax.experimental import pallas as pl
from jax.experimental.pallas import tpu as pltpu
from jax.experimental.pallas import tpu_sc as plsc

_N = 4096
_D = 4096
_DIM = 8192
_NC = 2
_NS = 16
_NW = _NC * _NS
_ROWS_PER_W = _N // _NW
_R = 4
_NCHUNK = _ROWS_PER_W // _R
_L = 16

_mesh = plsc.VectorSubcoreMesh(core_axis_name="c", subcore_axis_name="s")


@functools.partial(
    pl.kernel,
    mesh=_mesh,
    out_type=jax.ShapeDtypeStruct((_N, _D), jnp.float32),
    scratch_types=[
        pltpu.VMEM((_L,), jnp.int32),
        pltpu.VMEM((_L,), jnp.float32),
        pltpu.VMEM((_L,), jnp.float32),
        pltpu.VMEM((_L,), jnp.float32),
        pltpu.VMEM((_D,), jnp.float32),
        pltpu.VMEM((2, _R, _D), jnp.float32),
        pltpu.VMEM((2, _R, _D), jnp.float32),
        pltpu.SemaphoreType.DMA,
        pltpu.SemaphoreType.DMA,
        pltpu.SemaphoreType.DMA,
        pltpu.SemaphoreType.DMA,
        pltpu.SemaphoreType.DMA,
    ],
)
def _sc_kernel(x_hbm, idx_hbm, w_hbm, b_hbm, dv_hbm, cache_hbm, out_hbm,
               idx_v, dv_v, w_v, b_v, c_v, in_v, out_v,
               sem_p, isem0, isem1, osem0, osem1):
    wid = lax.axis_index("s") * _NC + lax.axis_index("c")
    row0 = wid * _ROWS_PER_W

    pltpu.sync_copy(idx_hbm, idx_v)
    pltpu.sync_copy(dv_hbm, dv_v)
    pltpu.sync_copy(cache_hbm, c_v)
    pltpu.async_copy(w_hbm.at[idx_v], w_v, sem_p).wait()
    pltpu.async_copy(b_hbm.at[idx_v], b_v, sem_p).wait()

    w = w_v[...]
    b = b_v[...]
    dv = jnp.clip(dv_v[...], 0.9, 1.0)

    def _cbody(g, carry):
        off = pl.multiple_of(g * _L, _L)
        c_v[pl.ds(off, _L)] = dv * c_v[pl.ds(off, _L)] + b
        return carry
    lax.fori_loop(0, _D // _L, _cbody, 0)

    isems = (isem0, isem1)
    osems = (osem0, osem1)

    def start_in(k):
        return pltpu.async_copy(
            x_hbm.at[pl.ds(row0 + k * _R, _R)], in_v.at[k % 2], isems[k % 2])

    def start_out(k):
        return pltpu.async_copy(
            out_v.at[k % 2], out_hbm.at[pl.ds(row0 + k * _R, _R)], osems[k % 2])

    def compute_chunk(k):
        buf_i = in_v.at[k % 2]
        buf_o = out_v.at[k % 2]

        def _body(cg, carry):
            off = pl.multiple_of(cg * _L, _L)
            cvec = c_v[pl.ds(off, _L)]
            for r in range(_R):
                buf_o[r, pl.ds(off, _L)] = w * buf_i[r, pl.ds(off, _L)] + cvec
            return carry
        lax.fori_loop(0, _D // _L, _body, 0)

    h_in = {}
    h_out = {}
    h_in[0] = start_in(0)
    for k in range(_NCHUNK):
        if k + 1 < _NCHUNK:
            h_in[k + 1] = start_in(k + 1)
        h_in[k].wait()
        if k >= 2:
            h_out[k - 2].wait()
        compute_chunk(k)
        h_out[k] = start_out(k)
    h_out[_NCHUNK - 2].wait()
    h_out[_NCHUNK - 1].wait()


def kernel(x, index, weight, bias, decay_value, cache):
    idx16 = jnp.full((_L,), index, jnp.int32)
    dv16 = jnp.broadcast_to(decay_value.astype(jnp.float32), (_L,))
    return _sc_kernel(x, idx16, weight.reshape(_DIM), bias, dv16, cache)


# speedup vs baseline: 1.0278x; 1.0278x over previous
import functools

import jax
import jax.numpy as jnp
from jax import lax
from jax.experimental import pallas as pl
from jax.experimental.pallas import tpu as pltpu
from jax.experimental.pallas import tpu_sc as plsc

_N = 4096
_D = 4096
_DIM = 8192
_NC = 2
_NS = 16
_NW = _NC * _NS
_ROWS_PER_W = _N // _NW
_R = 4
_NCHUNK = _ROWS_PER_W // _R
_L = 16

_mesh = plsc.VectorSubcoreMesh(core_axis_name="c", subcore_axis_name="s")


@functools.partial(
    pl.kernel,
    mesh=_mesh,
    out_type=jax.ShapeDtypeStruct((_N, _D), jnp.float32),
    scratch_types=[
        pltpu.VMEM((_L,), jnp.int32),
        pltpu.VMEM((_L,), jnp.float32),
        pltpu.VMEM((_L,), jnp.float32),
        pltpu.VMEM((_L,), jnp.float32),
        pltpu.VMEM((_D,), jnp.float32),
        pltpu.VMEM((2, _R, _D), jnp.float32),
        pltpu.VMEM((2, _R, _D), jnp.float32),
        pltpu.SemaphoreType.DMA,
        pltpu.SemaphoreType.DMA,
        pltpu.SemaphoreType.DMA,
        pltpu.SemaphoreType.DMA,
        pltpu.SemaphoreType.DMA,
    ],
)
def _sc_kernel(x_hbm, idx_hbm, w_hbm, b_hbm, dv_hbm, cache_hbm, out_hbm,
               idx_v, dv_v, w_v, b_v, c_v, in_v, out_v,
               sem_p, isem0, isem1, osem0, osem1):
    wid = lax.axis_index("s") * _NC + lax.axis_index("c")
    row0 = wid * _ROWS_PER_W

    pltpu.sync_copy(idx_hbm, idx_v)
    pltpu.sync_copy(dv_hbm, dv_v)
    pltpu.sync_copy(cache_hbm, c_v)
    pltpu.async_copy(w_hbm.at[idx_v], w_v, sem_p).wait()
    pltpu.async_copy(b_hbm.at[idx_v], b_v, sem_p).wait()

    w = w_v[...]
    b = b_v[...]
    dv = jnp.clip(dv_v[...], 0.9, 1.0)

    def _cbody(g, carry):
        off = pl.multiple_of(g * _L, _L)
        c_v[pl.ds(off, _L)] = dv * c_v[pl.ds(off, _L)] + b
        return carry
    lax.fori_loop(0, _D // _L, _cbody, 0)

    isems = (isem0, isem1)
    osems = (osem0, osem1)

    def start_in(k):
        return pltpu.async_copy(
            x_hbm.at[pl.ds(row0 + k * _R, _R)], in_v.at[k % 2], isems[k % 2])

    def start_out(k):
        return pltpu.async_copy(
            out_v.at[k % 2], out_hbm.at[pl.ds(row0 + k * _R, _R)], osems[k % 2])

    def compute_chunk(k):
        buf_i = in_v.at[k % 2]
        buf_o = out_v.at[k % 2]

        def _body(s, carry):
            off = pl.multiple_of(s * 128, 128)
            cregs = [c_v[pl.ds(off + g * _L, _L)] for g in range(8)]
            for r in range(_R):
                for g in range(8):
                    o2 = off + g * _L
                    buf_o[r, pl.ds(o2, _L)] = w * buf_i[r, pl.ds(o2, _L)] + cregs[g]
            return carry
        lax.fori_loop(0, _D // 128, _body, 0)

    h_in = {}
    h_out = {}
    h_in[0] = start_in(0)
    for k in range(_NCHUNK):
        if k + 1 < _NCHUNK:
            h_in[k + 1] = start_in(k + 1)
        h_in[k].wait()
        if k >= 2:
            h_out[k - 2].wait()
        compute_chunk(k)
        h_out[k] = start_out(k)
    h_out[_NCHUNK - 2].wait()
    h_out[_NCHUNK - 1].wait()


def kernel(x, index, weight, bias, decay_value, cache):
    idx16 = jnp.full((_L,), index, jnp.int32)
    dv16 = jnp.broadcast_to(decay_value.astype(jnp.float32), (_L,))
    return _sc_kernel(x, idx16, weight.reshape(_DIM), bias, dv16, cache)


# speedup vs baseline: 1.0703x; 1.0413x over previous
import functools

import jax
import jax.numpy as jnp
from jax import lax
from jax.experimental import pallas as pl
from jax.experimental.pallas import tpu as pltpu
from jax.experimental.pallas import tpu_sc as plsc

_N = 4096
_D = 4096
_DIM = 8192
_NC = 2
_NS = 16
_NW = _NC * _NS
_ROWS_PER_W = _N // _NW
_R = 4
_NCHUNK = _ROWS_PER_W // _R
_NBUF = 3
_L = 16

_mesh = plsc.VectorSubcoreMesh(core_axis_name="c", subcore_axis_name="s")


@functools.partial(
    pl.kernel,
    mesh=_mesh,
    out_type=jax.ShapeDtypeStruct((_N, _D), jnp.float32),
    scratch_types=[
        pltpu.VMEM((_L,), jnp.int32),
        pltpu.VMEM((_L,), jnp.float32),
        pltpu.VMEM((_L,), jnp.float32),
        pltpu.VMEM((_L,), jnp.float32),
        pltpu.VMEM((_D,), jnp.float32),
        pltpu.VMEM((_NBUF, _R, _D), jnp.float32),
        pltpu.VMEM((_NBUF, _R, _D), jnp.float32),
        pltpu.SemaphoreType.DMA,
        pltpu.SemaphoreType.DMA,
        pltpu.SemaphoreType.DMA,
        pltpu.SemaphoreType.DMA,
        pltpu.SemaphoreType.DMA,
        pltpu.SemaphoreType.DMA,
        pltpu.SemaphoreType.DMA,
    ],
)
def _sc_kernel(x_hbm, idx_hbm, w_hbm, b_hbm, dv_hbm, cache_hbm, out_hbm,
               idx_v, dv_v, w_v, b_v, c_v, in_v, out_v,
               sem_p, isem0, isem1, isem2, osem0, osem1, osem2):
    wid = lax.axis_index("s") * _NC + lax.axis_index("c")
    row0 = wid * _ROWS_PER_W

    isems = (isem0, isem1, isem2)
    osems = (osem0, osem1, osem2)

    def start_in(k):
        return pltpu.async_copy(
            x_hbm.at[pl.ds(row0 + k * _R, _R)], in_v.at[k % _NBUF],
            isems[k % _NBUF])

    def start_out(k):
        return pltpu.async_copy(
            out_v.at[k % _NBUF], out_hbm.at[pl.ds(row0 + k * _R, _R)],
            osems[k % _NBUF])

    h_in = {}
    for k in range(_NBUF):
        h_in[k] = start_in(k)

    pltpu.sync_copy(idx_hbm, idx_v)
    pltpu.sync_copy(dv_hbm, dv_v)
    pltpu.sync_copy(cache_hbm, c_v)
    pltpu.async_copy(w_hbm.at[idx_v], w_v, sem_p).wait()
    pltpu.async_copy(b_hbm.at[idx_v], b_v, sem_p).wait()

    w = w_v[...]
    b = b_v[...]
    dv = jnp.clip(dv_v[...], 0.9, 1.0)

    def _cbody(g, carry):
        off = pl.multiple_of(g * _L, _L)
        c_v[pl.ds(off, _L)] = dv * c_v[pl.ds(off, _L)] + b
        return carry
    lax.fori_loop(0, _D // _L, _cbody, 0)

    def compute_chunk(k):
        buf_i = in_v.at[k % _NBUF]
        buf_o = out_v.at[k % _NBUF]

        def _body(s, carry):
            off = pl.multiple_of(s * 128, 128)
            cregs = [c_v[pl.ds(off + g * _L, _L)] for g in range(8)]
            for r in range(_R):
                for g in range(8):
                    o2 = off + g * _L
                    buf_o[r, pl.ds(o2, _L)] = w * buf_i[r, pl.ds(o2, _L)] + cregs[g]
            return carry
        lax.fori_loop(0, _D // 128, _body, 0)

    h_out = {}
    for k in range(_NCHUNK):
        h_in[k].wait()
        if k >= _NBUF:
            h_out[k - _NBUF].wait()
        compute_chunk(k)
        h_out[k] = start_out(k)
        if k + _NBUF < _NCHUNK:
            h_in[k + _NBUF] = start_in(k + _NBUF)
    for k in range(_NCHUNK - _NBUF, _NCHUNK):
        h_out[k].wait()


def kernel(x, index, weight, bias, decay_value, cache):
    idx16 = jnp.full((_L,), index, jnp.int32)
    dv16 = jnp.broadcast_to(decay_value.astype(jnp.float32), (_L,))
    return _sc_kernel(x, idx16, weight.reshape(_DIM), bias, dv16, cache)


# speedup vs baseline: 1.1016x; 1.0292x over previous
import functools

import jax
import jax.numpy as jnp
from jax import lax
from jax.experimental import pallas as pl
from jax.experimental.pallas import tpu as pltpu
from jax.experimental.pallas import tpu_sc as plsc

_N = 4096
_D = 4096
_DIM = 8192
_NC = 2
_NS = 16
_NW = _NC * _NS
_ROWS_PER_W = _N // _NW
_R = 4
_NCHUNK = _ROWS_PER_W // _R
_NBUF = 6
_L = 16

_mesh = plsc.VectorSubcoreMesh(core_axis_name="c", subcore_axis_name="s")


@functools.partial(
    pl.kernel,
    mesh=_mesh,
    out_type=jax.ShapeDtypeStruct((_N, _D), jnp.float32),
    scratch_types=[
        pltpu.VMEM((_L,), jnp.int32),
        pltpu.VMEM((_L,), jnp.float32),
        pltpu.VMEM((_L,), jnp.float32),
        pltpu.VMEM((_L,), jnp.float32),
        pltpu.VMEM((_D,), jnp.float32),
        pltpu.VMEM((_NBUF, _R, _D), jnp.float32),
        pltpu.SemaphoreType.DMA,
        pltpu.SemaphoreType.DMA,
        pltpu.SemaphoreType.DMA,
        pltpu.SemaphoreType.DMA,
        pltpu.SemaphoreType.DMA,
        pltpu.SemaphoreType.DMA,
        pltpu.SemaphoreType.DMA,
        pltpu.SemaphoreType.DMA,
        pltpu.SemaphoreType.DMA,
        pltpu.SemaphoreType.DMA,
        pltpu.SemaphoreType.DMA,
        pltpu.SemaphoreType.DMA,
        pltpu.SemaphoreType.DMA,
    ],
)
def _sc_kernel(x_hbm, idx_hbm, w_hbm, b_hbm, dv_hbm, cache_hbm, out_hbm,
               idx_v, dv_v, w_v, b_v, c_v, ring_v,
               sem_p, isem0, isem1, isem2, isem3, isem4, isem5,
               osem0, osem1, osem2, osem3, osem4, osem5):
    wid = lax.axis_index("s") * _NC + lax.axis_index("c")
    row0 = wid * _ROWS_PER_W

    isems = (isem0, isem1, isem2, isem3, isem4, isem5)
    osems = (osem0, osem1, osem2, osem3, osem4, osem5)

    def start_in(k):
        return pltpu.async_copy(
            x_hbm.at[pl.ds(row0 + k * _R, _R)], ring_v.at[k % _NBUF],
            isems[k % _NBUF])

    def start_out(k):
        return pltpu.async_copy(
            ring_v.at[k % _NBUF], out_hbm.at[pl.ds(row0 + k * _R, _R)],
            osems[k % _NBUF])

    h_in = {}
    for k in range(3):
        h_in[k] = start_in(k)

    pltpu.sync_copy(idx_hbm, idx_v)
    pltpu.sync_copy(dv_hbm, dv_v)
    pltpu.sync_copy(cache_hbm, c_v)
    pltpu.async_copy(w_hbm.at[idx_v], w_v, sem_p).wait()
    pltpu.async_copy(b_hbm.at[idx_v], b_v, sem_p).wait()

    w = w_v[...]
    b = b_v[...]
    dv = jnp.clip(dv_v[...], 0.9, 1.0)

    def _cbody(g, carry):
        off = pl.multiple_of(g * _L, _L)
        c_v[pl.ds(off, _L)] = dv * c_v[pl.ds(off, _L)] + b
        return carry
    lax.fori_loop(0, _D // _L, _cbody, 0)

    def compute_chunk(k):
        buf = ring_v.at[k % _NBUF]

        def _body(s, carry):
            off = pl.multiple_of(s * 128, 128)
            cregs = [c_v[pl.ds(off + g * _L, _L)] for g in range(8)]
            for r in range(_R):
                for g in range(8):
                    o2 = off + g * _L
                    buf[r, pl.ds(o2, _L)] = w * buf[r, pl.ds(o2, _L)] + cregs[g]
            return carry
        lax.fori_loop(0, _D // 128, _body, 0)

    h_out = {}
    waited = 0
    for k in range(_NCHUNK):
        h_in[k].wait()
        compute_chunk(k)
        h_out[k] = start_out(k)
        if k + 3 < _NCHUNK:
            if k >= 3:
                h_out[k - 3].wait()
                waited = k - 3 + 1
            h_in[k + 3] = start_in(k + 3)
    for k in range(waited, _NCHUNK):
        h_out[k].wait()


def kernel(x, index, weight, bias, decay_value, cache):
    idx16 = jnp.full((_L,), index, jnp.int32)
    dv16 = jnp.broadcast_to(decay_value.astype(jnp.float32), (_L,))
    return _sc_kernel(x, idx16, weight.reshape(_DIM), bias, dv16, cache)


# speedup vs baseline: 1.1077x; 1.0056x over previous
import functools

import jax
import jax.numpy as jnp
from jax import lax
from jax.experimental import pallas as pl
from jax.experimental.pallas import tpu as pltpu
from jax.experimental.pallas import tpu_sc as plsc

_N = 4096
_D = 4096
_DIM = 8192
_NC = 2
_NS = 16
_NW = _NC * _NS
_ROWS_PER_W = _N // _NW
_R = 4
_NCHUNK = _ROWS_PER_W // _R
_NBUF = 6
_L = 16

_mesh = plsc.VectorSubcoreMesh(core_axis_name="c", subcore_axis_name="s")


@functools.partial(
    pl.kernel,
    mesh=_mesh,
    out_type=jax.ShapeDtypeStruct((_N, _D), jnp.float32),
    scratch_types=[
        pltpu.VMEM((_L,), jnp.int32),
        pltpu.VMEM((_L,), jnp.float32),
        pltpu.VMEM((_L,), jnp.float32),
        pltpu.VMEM((_L,), jnp.float32),
        pltpu.VMEM((_D,), jnp.float32),
        pltpu.VMEM((_NBUF, _R, _D), jnp.float32),
        pltpu.SemaphoreType.DMA,
        pltpu.SemaphoreType.DMA,
        pltpu.SemaphoreType.DMA,
        pltpu.SemaphoreType.DMA,
        pltpu.SemaphoreType.DMA,
        pltpu.SemaphoreType.DMA,
        pltpu.SemaphoreType.DMA,
        pltpu.SemaphoreType.DMA,
        pltpu.SemaphoreType.DMA,
        pltpu.SemaphoreType.DMA,
        pltpu.SemaphoreType.DMA,
        pltpu.SemaphoreType.DMA,
        pltpu.SemaphoreType.DMA,
    ],
)
def _sc_kernel(x_hbm, idx_hbm, w_hbm, b_hbm, dv_hbm, cache_hbm, out_hbm,
               idx_v, dv_v, w_v, b_v, c_v, ring_v,
               sem_p, isem0, isem1, isem2, isem3, isem4, isem5,
               osem0, osem1, osem2, osem3, osem4, osem5):
    wid = lax.axis_index("s") * _NC + lax.axis_index("c")
    row0 = wid * _ROWS_PER_W

    isems = (isem0, isem1, isem2, isem3, isem4, isem5)
    osems = (osem0, osem1, osem2, osem3, osem4, osem5)

    def start_in(k):
        return pltpu.async_copy(
            x_hbm.at[pl.ds(row0 + k * _R, _R)], ring_v.at[k % _NBUF],
            isems[k % _NBUF])

    def start_out(k):
        return pltpu.async_copy(
            ring_v.at[k % _NBUF], out_hbm.at[pl.ds(row0 + k * _R, _R)],
            osems[k % _NBUF])

    h_in = {}
    for k in range(4):
        h_in[k] = start_in(k)

    h_idx = pltpu.async_copy(idx_hbm, idx_v, osems[0])
    h_dv = pltpu.async_copy(dv_hbm, dv_v, osems[1])
    h_cache = pltpu.async_copy(cache_hbm, c_v, osems[2])
    h_idx.wait()
    h_w = pltpu.async_copy(w_hbm.at[idx_v], w_v, sem_p)
    h_b = pltpu.async_copy(b_hbm.at[idx_v], b_v, osems[3])
    h_w.wait()
    h_b.wait()
    h_dv.wait()
    h_cache.wait()

    w = w_v[...]
    b = b_v[...]
    dv = jnp.clip(dv_v[...], 0.9, 1.0)

    def _cbody(g, carry):
        off = pl.multiple_of(g * _L, _L)
        c_v[pl.ds(off, _L)] = dv * c_v[pl.ds(off, _L)] + b
        return carry
    lax.fori_loop(0, _D // _L, _cbody, 0)

    def compute_chunk(k):
        buf = ring_v.at[k % _NBUF]

        def _body(s, carry):
            off = pl.multiple_of(s * 256, 256)
            cregs = [c_v[pl.ds(off + g * _L, _L)] for g in range(16)]
            for r in range(_R):
                for g in range(16):
                    o2 = off + g * _L
                    buf[r, pl.ds(o2, _L)] = w * buf[r, pl.ds(o2, _L)] + cregs[g]
            return carry
        lax.fori_loop(0, _D // 256, _body, 0)

    h_out = {}
    waited = 0
    for k in range(_NCHUNK):
        h_in[k].wait()
        compute_chunk(k)
        h_out[k] = start_out(k)
        if k + 4 < _NCHUNK:
            if k >= 2:
                h_out[k - 2].wait()
                waited = k - 2 + 1
            h_in[k + 4] = start_in(k + 4)
    for k in range(waited, _NCHUNK):
        h_out[k].wait()


def kernel(x, index, weight, bias, decay_value, cache):
    idx16 = jnp.full((_L,), index, jnp.int32)
    dv16 = jnp.broadcast_to(decay_value.astype(jnp.float32), (_L,))
    return _sc_kernel(x, idx16, weight.reshape(_DIM), bias, dv16, cache)
